# Initial kernel scaffold; baseline (speedup 1.0000x reference)
import jax
import jax.numpy as jnp
from jax.experimental import pallas as pl


def kernel(x, edge_index, norm_distance, init_edge_states, Wm, bm, imp_mask, fi_W1, fi_b1, fi_g1, fi_be1, fi_W2, fi_b2, fi_g2, fi_be2, fi_W3, fi_b3, fu_W1, fu_b1, fu_g1, fu_be1, fu_W2, fu_b2):
    raise NotImplementedError("write your pallas kernel here")


# speedup vs baseline: 2.5839x; 2.5839x over previous
import functools

import jax
import jax.numpy as jnp
from jax import lax
from jax.experimental import pallas as pl
from jax.experimental.pallas import tpu as pltpu
from jax.experimental.pallas import tpu_sc as plsc

_EPS = 1e-5
_INTERPRET = False

_NC = 2
_NS = 16
_CK = 128


def _leaky(v):
    return jnp.where(v >= 0, v, 0.01 * v)


def _gather_body(src_hbm, dst_hbm, x_hbm, xsrc_hbm, xdst_hbm,
                 idx_a, idx_b, rows_a, rows_b, sem_a, sem_b):
    E = src_hbm.shape[0]
    nchunks = E // _CK
    c = lax.axis_index("c")
    s = lax.axis_index("s")
    wid = s * _NC + c
    nw = _NC * _NS
    iters = pl.cdiv(nchunks, nw)

    def body(k, _):
        chunk = k * nw + wid

        @pl.when(chunk < nchunks)
        def _():
            base = chunk * _CK
            pltpu.sync_copy(src_hbm.at[pl.ds(base, _CK)], idx_a)
            pltpu.sync_copy(dst_hbm.at[pl.ds(base, _CK)], idx_b)
            cp_a = pltpu.async_copy(x_hbm.at[idx_a], rows_a, sem_a)
            cp_b = pltpu.async_copy(x_hbm.at[idx_b], rows_b, sem_b)
            cp_a.wait()
            pltpu.sync_copy(rows_a, xsrc_hbm.at[pl.ds(base, _CK)])
            cp_b.wait()
            pltpu.sync_copy(rows_b, xdst_hbm.at[pl.ds(base, _CK)])

        return 0

    lax.fori_loop(0, iters, body, 0)


def _sc_gather(x, src, dst):
    E = src.shape[0]
    H = x.shape[1]
    mesh = plsc.VectorSubcoreMesh(core_axis_name="c", subcore_axis_name="s")
    f = pl.kernel(
        _gather_body,
        out_type=(
            jax.ShapeDtypeStruct((E, H), jnp.float32),
            jax.ShapeDtypeStruct((E, H), jnp.float32),
        ),
        mesh=mesh,
        scratch_types=[
            pltpu.VMEM((_CK,), jnp.int32),
            pltpu.VMEM((_CK,), jnp.int32),
            pltpu.VMEM((_CK, H), jnp.float32),
            pltpu.VMEM((_CK, H), jnp.float32),
            pltpu.SemaphoreType.DMA,
            pltpu.SemaphoreType.DMA,
        ],
    )
    return f(src, dst, x)


def _scatter_body(dst_hbm, mst_hbm, out_hbm, idx_v, rows_v, acc):
    E = dst_hbm.shape[0]
    N = acc.shape[0]
    c = lax.axis_index("c")
    s = lax.axis_index("s")
    e_half = E // _NC
    nchunks = e_half // _CK
    iters = pl.cdiv(nchunks, _NS)

    ntiles_z = 10
    rows_per_tile = N // ntiles_z
    zrows = rows_v.shape[0]

    def zero_buf(r, _):
        for j in range(8):
            rows_v[r, pl.ds(j * 16, 16)] = jnp.zeros((16,), jnp.float32)
        return 0

    lax.fori_loop(0, zrows, zero_buf, 0)

    nz = rows_per_tile // zrows
    rem = rows_per_tile - nz * zrows

    @pl.when(s < ntiles_z)
    def _():
        def zero_acc(q, _):
            pltpu.sync_copy(rows_v,
                            acc.at[pl.ds(s * rows_per_tile + q * zrows, zrows)])
            return 0

        lax.fori_loop(0, nz, zero_acc, 0)
        if rem:
            pltpu.sync_copy(rows_v.at[pl.ds(0, rem)],
                            acc.at[pl.ds(s * rows_per_tile + nz * zrows, rem)])

    plsc.subcore_barrier()

    def body(k, _):
        chunk = k * _NS + s

        @pl.when(chunk < nchunks)
        def _():
            base = c * e_half + chunk * _CK
            pltpu.sync_copy(dst_hbm.at[pl.ds(base, _CK)], idx_v)
            pltpu.sync_copy(mst_hbm.at[pl.ds(base, _CK)], rows_v)
            pltpu.sync_copy(rows_v, acc.at[idx_v], add=True)

        return 0

    lax.fori_loop(0, iters, body, 0)

    plsc.subcore_barrier()

    @pl.when(s < ntiles_z)
    def _():
        pltpu.sync_copy(acc.at[pl.ds(s * rows_per_tile, rows_per_tile)],
                        out_hbm.at[c, pl.ds(s * rows_per_tile, rows_per_tile)])


def _sc_scatter(m_st, dst, N):
    E, H = m_st.shape
    mesh = plsc.VectorSubcoreMesh(core_axis_name="c", subcore_axis_name="s")
    f = pl.kernel(
        _scatter_body,
        out_type=jax.ShapeDtypeStruct((_NC, N, H), jnp.float32),
        mesh=mesh,
        scratch_types=[
            pltpu.VMEM((_CK,), jnp.int32),
            pltpu.VMEM((_CK, H), jnp.float32),
            pltpu.VMEM_SHARED((N, H), jnp.float32),
        ],
    )
    return f(dst, m_st)


def _decay_body(nd_ref, out_ref):
    t = (jnp.pi / 2.0) * nd_ref[...]
    t2 = t * t
    c = 1.0 / 479001600.0
    for k in (-1.0 / 3628800.0, 1.0 / 40320.0, -1.0 / 720.0, 1.0 / 24.0,
              -0.5, 1.0):
        c = c * t2 + k
    out_ref[...] = c


def _decay(norm_distance):
    E = norm_distance.shape[0]
    nd2 = norm_distance.reshape(E // 128, 128)
    out = pl.pallas_call(
        _decay_body,
        out_shape=jax.ShapeDtypeStruct(nd2.shape, jnp.float32),
        interpret=_INTERPRET,
    )(nd2)
    return out.reshape(E, 1)


_BE = 2560


def _passA_body(xsrc_ref, es_ref, xdst_ref, W1_ref, b1_ref,
                a1_ref, st1_ref, acc):
    i = pl.program_id(0)

    @pl.when(i == 0)
    def _():
        acc[...] = jnp.zeros_like(acc)

    h = jnp.concatenate([xsrc_ref[...], es_ref[...], xdst_ref[...]], axis=1)
    p = jnp.dot(h, W1_ref[...], preferred_element_type=jnp.float32) + b1_ref[0:1, :]
    a = _leaky(p)
    a1_ref[...] = a
    acc[0:1, :] += jnp.sum(a, axis=0, keepdims=True)
    acc[1:2, :] += jnp.sum(a * a, axis=0, keepdims=True)

    @pl.when(i == pl.num_programs(0) - 1)
    def _():
        st1_ref[...] = acc[...]


def _bn_scale(st_ref, g_ref, be_ref, count):
    m = st_ref[0:1, :] / count
    v = st_ref[1:2, :] / count - m * m
    s = g_ref[...] * lax.rsqrt(v + _EPS)
    t = be_ref[...] - m * s
    return s, t


def _passB_body(E, a1_ref, st1_ref, W2_ref, b2_ref, g1_ref, be1_ref,
                st2_ref, acc):
    i = pl.program_id(0)

    @pl.when(i == 0)
    def _():
        acc[...] = jnp.zeros_like(acc)

    s1, t1 = _bn_scale(st1_ref, g1_ref, be1_ref, E)
    a1n = a1_ref[...] * s1 + t1
    a2 = _leaky(jnp.dot(a1n, W2_ref[...], preferred_element_type=jnp.float32)
                + b2_ref[0:1, :])
    acc[0:1, :] += jnp.sum(a2, axis=0, keepdims=True)
    acc[1:2, :] += jnp.sum(a2 * a2, axis=0, keepdims=True)

    @pl.when(i == pl.num_programs(0) - 1)
    def _():
        st2_ref[...] = acc[...]


def _passC_body(E, a1_ref, xsrc_ref, nd_ref, st1_ref, st2_ref,
                W2_ref, b2_ref, g1_ref, be1_ref,
                W3_ref, b3_ref, g2_ref, be2_ref,
                Wm_ref, bm_ref, out_ref):
    s1, t1 = _bn_scale(st1_ref, g1_ref, be1_ref, E)
    s2, t2 = _bn_scale(st2_ref, g2_ref, be2_ref, E)
    a1n = a1_ref[...] * s1 + t1
    a2 = _leaky(jnp.dot(a1n, W2_ref[...], preferred_element_type=jnp.float32)
                + b2_ref[0:1, :])
    a2n = a2 * s2 + t2
    h3 = jnp.dot(a2n, W3_ref[...], preferred_element_type=jnp.float32) + b3_ref[0:1, :]
    G = jnp.dot(xsrc_ref[...], Wm_ref[...], preferred_element_type=jnp.float32) + bm_ref[0:1, :]
    out_ref[...] = nd_ref[...] * h3 * G


def _row_block(i):
    return (i, 0)


def _pinned(i):
    return (0, 0)


def _edge_passes(xsrc, xdst, es, nd, fi_W1, fi_b1, fi_g1, fi_be1,
                 fi_W2, fi_b2, fi_g2, fi_be2, fi_W3, fi_b3, Wm, bm):
    E, H = xsrc.shape
    nb = E // _BE
    row = pl.BlockSpec((_BE, H), _row_block)
    row1 = pl.BlockSpec((_BE, 1), _row_block)
    full = lambda shape: pl.BlockSpec(shape, _pinned)
    st_shape = jax.ShapeDtypeStruct((8, H), jnp.float32)
    vec = lambda a: a.reshape(1, H)

    a1, st1 = pl.pallas_call(
        _passA_body,
        grid=(nb,),
        in_specs=[row, row, row, full((3 * H, H)), full((1, H))],
        out_specs=[row, full((8, H))],
        out_shape=[jax.ShapeDtypeStruct((E, H), jnp.float32), st_shape],
        scratch_shapes=[pltpu.VMEM((8, H), jnp.float32)],
        interpret=_INTERPRET,
    )(xsrc, es, xdst, fi_W1, vec(fi_b1))

    (st2,) = pl.pallas_call(
        functools.partial(_passB_body, float(E)),
        grid=(nb,),
        in_specs=[row, full((8, H)), full((H, H)), full((1, H)),
                  full((1, H)), full((1, H))],
        out_specs=[full((8, H))],
        out_shape=[st_shape],
        scratch_shapes=[pltpu.VMEM((8, H), jnp.float32)],
        interpret=_INTERPRET,
    )(a1, st1, fi_W2, vec(fi_b2), vec(fi_g1), vec(fi_be1))

    m_st = pl.pallas_call(
        functools.partial(_passC_body, float(E)),
        grid=(nb,),
        in_specs=[row, row, row1, full((8, H)), full((8, H)),
                  full((H, H)), full((1, H)), full((1, H)), full((1, H)),
                  full((H, H)), full((1, H)), full((1, H)), full((1, H)),
                  full((H, H)), full((1, H))],
        out_specs=row,
        out_shape=jax.ShapeDtypeStruct((E, H), jnp.float32),
        interpret=_INTERPRET,
    )(a1, xsrc, nd, st1, st2,
      fi_W2, vec(fi_b2), vec(fi_g1), vec(fi_be1),
      fi_W3, vec(fi_b3), vec(fi_g2), vec(fi_be2),
      Wm, vec(bm))
    return m_st


def _node_body(x_ref, inc_ref, Wm_ref, bm_ref, imp_ref,
               W1_ref, b1_ref, g1_ref, be1_ref, W2_ref, b2_ref, out_ref):
    x = x_ref[...]
    n = x.shape[0]
    inc = inc_ref[0] + inc_ref[1]
    mt = imp_ref[...] * (jnp.dot(x, Wm_ref[...], preferred_element_type=jnp.float32)
                         + bm_ref[0:1, :])
    u = mt + inc
    t = _leaky(jnp.dot(u, W1_ref[...], preferred_element_type=jnp.float32)
               + b1_ref[0:1, :])
    m = jnp.sum(t, axis=0, keepdims=True) / n
    v = jnp.sum(t * t, axis=0, keepdims=True) / n - m * m
    t = (t - m) * lax.rsqrt(v + _EPS) * g1_ref[...] + be1_ref[...]
    out_ref[...] = (jnp.dot(t, W2_ref[...], preferred_element_type=jnp.float32)
                    + b2_ref[0:1, :] + x)


def _node_finale(x, parts, Wm, bm, imp_mask, fu_W1, fu_b1, fu_g1, fu_be1,
                 fu_W2, fu_b2):
    N, H = x.shape
    vec = lambda a: a.reshape(1, H)
    return pl.pallas_call(
        _node_body,
        out_shape=jax.ShapeDtypeStruct((N, H), jnp.float32),
        interpret=_INTERPRET,
    )(x, parts, Wm, vec(bm), imp_mask, fu_W1, vec(fu_b1), vec(fu_g1),
      vec(fu_be1), fu_W2, vec(fu_b2))


def kernel(x, edge_index, norm_distance, init_edge_states, Wm, bm, imp_mask,
           fi_W1, fi_b1, fi_g1, fi_be1, fi_W2, fi_b2, fi_g2, fi_be2,
           fi_W3, fi_b3, fu_W1, fu_b1, fu_g1, fu_be1, fu_W2, fu_b2):
    N, H = x.shape
    E = edge_index.shape[1]
    src = edge_index[0]
    dst = edge_index[1]
    dec = _decay(norm_distance)

    xsrc, xdst = _sc_gather(x, src, dst)
    m_st = _edge_passes(xsrc, xdst, init_edge_states, dec,
                        fi_W1, fi_b1, fi_g1, fi_be1,
                        fi_W2, fi_b2, fi_g2, fi_be2,
                        fi_W3, fi_b3, Wm, bm)
    parts = _sc_scatter(m_st, dst, N)
    return _node_finale(x, parts, Wm, bm, imp_mask,
                        fu_W1, fu_b1, fu_g1, fu_be1, fu_W2, fu_b2)


# speedup vs baseline: 2.6416x; 1.0223x over previous
import functools

import jax
import jax.numpy as jnp
from jax import lax
from jax.experimental import pallas as pl
from jax.experimental.pallas import tpu as pltpu
from jax.experimental.pallas import tpu_sc as plsc

_EPS = 1e-5
_INTERPRET = False

_NC = 2
_NS = 16
_CK = 128


def _leaky(v):
    return jnp.where(v >= 0, v, 0.01 * v)


def _gather_body(src_hbm, dst_hbm, x_hbm, xsrc_hbm, xdst_hbm,
                 idx_a, idx_b, rows_a, rows_b, sem_a, sem_b):
    E = src_hbm.shape[0]
    nchunks = E // _CK
    c = lax.axis_index("c")
    s = lax.axis_index("s")
    wid = s * _NC + c
    nw = _NC * _NS
    iters = pl.cdiv(nchunks, nw)

    def body(k, _):
        chunk = k * nw + wid

        @pl.when(chunk < nchunks)
        def _():
            base = chunk * _CK
            pltpu.sync_copy(src_hbm.at[pl.ds(base, _CK)], idx_a)
            pltpu.sync_copy(dst_hbm.at[pl.ds(base, _CK)], idx_b)
            cp_a = pltpu.async_copy(x_hbm.at[idx_a], rows_a, sem_a)
            cp_b = pltpu.async_copy(x_hbm.at[idx_b], rows_b, sem_b)
            cp_a.wait()
            pltpu.sync_copy(rows_a, xsrc_hbm.at[pl.ds(base, _CK)])
            cp_b.wait()
            pltpu.sync_copy(rows_b, xdst_hbm.at[pl.ds(base, _CK)])

        return 0

    lax.fori_loop(0, iters, body, 0)


def _sc_gather(x, src, dst):
    E = src.shape[0]
    H = x.shape[1]
    mesh = plsc.VectorSubcoreMesh(core_axis_name="c", subcore_axis_name="s")
    f = pl.kernel(
        _gather_body,
        out_type=(
            jax.ShapeDtypeStruct((E, H), jnp.float32),
            jax.ShapeDtypeStruct((E, H), jnp.float32),
        ),
        mesh=mesh,
        scratch_types=[
            pltpu.VMEM((_CK,), jnp.int32),
            pltpu.VMEM((_CK,), jnp.int32),
            pltpu.VMEM((_CK, H), jnp.float32),
            pltpu.VMEM((_CK, H), jnp.float32),
            pltpu.SemaphoreType.DMA,
            pltpu.SemaphoreType.DMA,
        ],
    )
    return f(src, dst, x)


def _scatter_body(dst_hbm, mst_hbm, out_hbm, idx_v, rows_v, acc):
    E = dst_hbm.shape[0]
    N = acc.shape[0]
    c = lax.axis_index("c")
    s = lax.axis_index("s")
    e_half = E // _NC
    nchunks = e_half // _CK
    iters = pl.cdiv(nchunks, _NS)

    ntiles_z = 10
    rows_per_tile = N // ntiles_z
    zrows = rows_v.shape[0]

    def zero_buf(r, _):
        for j in range(8):
            rows_v[r, pl.ds(j * 16, 16)] = jnp.zeros((16,), jnp.float32)
        return 0

    lax.fori_loop(0, zrows, zero_buf, 0)

    nz = rows_per_tile // zrows
    rem = rows_per_tile - nz * zrows

    @pl.when(s < ntiles_z)
    def _():
        def zero_acc(q, _):
            pltpu.sync_copy(rows_v,
                            acc.at[pl.ds(s * rows_per_tile + q * zrows, zrows)])
            return 0

        lax.fori_loop(0, nz, zero_acc, 0)
        if rem:
            pltpu.sync_copy(rows_v.at[pl.ds(0, rem)],
                            acc.at[pl.ds(s * rows_per_tile + nz * zrows, rem)])

    plsc.subcore_barrier()

    def body(k, _):
        chunk = k * _NS + s

        @pl.when(chunk < nchunks)
        def _():
            base = c * e_half + chunk * _CK
            pltpu.sync_copy(dst_hbm.at[pl.ds(base, _CK)], idx_v)
            pltpu.sync_copy(mst_hbm.at[pl.ds(base, _CK)], rows_v)
            pltpu.sync_copy(rows_v, acc.at[idx_v], add=True)

        return 0

    lax.fori_loop(0, iters, body, 0)

    plsc.subcore_barrier()

    @pl.when(s < ntiles_z)
    def _():
        pltpu.sync_copy(acc.at[pl.ds(s * rows_per_tile, rows_per_tile)],
                        out_hbm.at[c, pl.ds(s * rows_per_tile, rows_per_tile)])


def _sc_scatter(m_st, dst, N):
    E, H = m_st.shape
    mesh = plsc.VectorSubcoreMesh(core_axis_name="c", subcore_axis_name="s")
    f = pl.kernel(
        _scatter_body,
        out_type=jax.ShapeDtypeStruct((_NC, N, H), jnp.float32),
        mesh=mesh,
        scratch_types=[
            pltpu.VMEM((_CK,), jnp.int32),
            pltpu.VMEM((_CK, H), jnp.float32),
            pltpu.VMEM_SHARED((N, H), jnp.float32),
        ],
    )
    return f(dst, m_st)


def _decay_body(nd_ref, out_ref):
    t = (jnp.pi / 2.0) * nd_ref[...]
    t2 = t * t
    c = 1.0 / 479001600.0
    for k in (-1.0 / 3628800.0, 1.0 / 40320.0, -1.0 / 720.0, 1.0 / 24.0,
              -0.5, 1.0):
        c = c * t2 + k
    out_ref[...] = c


def _decay(norm_distance):
    E = norm_distance.shape[0]
    nd2 = norm_distance.reshape(E // 128, 128)
    out = pl.pallas_call(
        _decay_body,
        out_shape=jax.ShapeDtypeStruct(nd2.shape, jnp.float32),
        interpret=_INTERPRET,
    )(nd2)
    return out.reshape(E, 1)


_BE = 2560


def _passA_body(xsrc_ref, es_ref, xdst_ref, W1_ref, b1_ref,
                a1_ref, st1_ref, acc):
    i = pl.program_id(0)

    @pl.when(i == 0)
    def _():
        acc[...] = jnp.zeros_like(acc)

    h = jnp.concatenate([xsrc_ref[...], es_ref[...], xdst_ref[...]],
                        axis=1).astype(jnp.bfloat16)
    p = jnp.dot(h, W1_ref[...].astype(jnp.bfloat16),
                preferred_element_type=jnp.float32) + b1_ref[0:1, :]
    a = _leaky(p)
    a1_ref[...] = a.astype(jnp.bfloat16)
    acc[0:1, :] += jnp.sum(a, axis=0, keepdims=True)
    acc[1:2, :] += jnp.sum(a * a, axis=0, keepdims=True)

    @pl.when(i == pl.num_programs(0) - 1)
    def _():
        st1_ref[...] = acc[...]


def _bn_scale(st_ref, g_ref, be_ref, count):
    m = st_ref[0:1, :] / count
    v = st_ref[1:2, :] / count - m * m
    s = g_ref[...] * lax.rsqrt(v + _EPS)
    t = be_ref[...] - m * s
    return s, t


def _folded_layer2(a1_bf, st1_ref, W2_ref, b2_ref, g1_ref, be1_ref, E):
    s1, t1 = _bn_scale(st1_ref, g1_ref, be1_ref, E)
    W2f = (jnp.transpose(s1) * W2_ref[...]).astype(jnp.bfloat16)
    bias = (jnp.dot(t1, W2_ref[...], preferred_element_type=jnp.float32)
            + b2_ref[0:1, :])
    return _leaky(jnp.dot(a1_bf, W2f, preferred_element_type=jnp.float32)
                  + bias)


def _passB_body(E, a1_ref, st1_ref, W2_ref, b2_ref, g1_ref, be1_ref,
                st2_ref, acc):
    i = pl.program_id(0)

    @pl.when(i == 0)
    def _():
        acc[...] = jnp.zeros_like(acc)

    a2 = _folded_layer2(a1_ref[...], st1_ref, W2_ref, b2_ref, g1_ref,
                        be1_ref, E)
    acc[0:1, :] += jnp.sum(a2, axis=0, keepdims=True)
    acc[1:2, :] += jnp.sum(a2 * a2, axis=0, keepdims=True)

    @pl.when(i == pl.num_programs(0) - 1)
    def _():
        st2_ref[...] = acc[...]


def _passC_body(E, a1_ref, xsrc_ref, nd_ref, st1_ref, st2_ref,
                W2_ref, b2_ref, g1_ref, be1_ref,
                W3_ref, b3_ref, g2_ref, be2_ref,
                Wm_ref, bm_ref, out_ref):
    a2 = _folded_layer2(a1_ref[...], st1_ref, W2_ref, b2_ref, g1_ref,
                        be1_ref, E)
    s2, t2 = _bn_scale(st2_ref, g2_ref, be2_ref, E)
    W3f = (jnp.transpose(s2) * W3_ref[...]).astype(jnp.bfloat16)
    bias3 = (jnp.dot(t2, W3_ref[...], preferred_element_type=jnp.float32)
             + b3_ref[0:1, :])
    h3 = jnp.dot(a2.astype(jnp.bfloat16), W3f,
                 preferred_element_type=jnp.float32) + bias3
    G = jnp.dot(xsrc_ref[...].astype(jnp.bfloat16),
                Wm_ref[...].astype(jnp.bfloat16),
                preferred_element_type=jnp.float32) + bm_ref[0:1, :]
    out_ref[...] = nd_ref[...] * h3 * G


def _row_block(i):
    return (i, 0)


def _pinned(i):
    return (0, 0)


def _edge_passes(xsrc, xdst, es, nd, fi_W1, fi_b1, fi_g1, fi_be1,
                 fi_W2, fi_b2, fi_g2, fi_be2, fi_W3, fi_b3, Wm, bm):
    E, H = xsrc.shape
    nb = E // _BE
    row = pl.BlockSpec((_BE, H), _row_block)
    row1 = pl.BlockSpec((_BE, 1), _row_block)
    full = lambda shape: pl.BlockSpec(shape, _pinned)
    st_shape = jax.ShapeDtypeStruct((8, H), jnp.float32)
    vec = lambda a: a.reshape(1, H)

    a1, st1 = pl.pallas_call(
        _passA_body,
        grid=(nb,),
        in_specs=[row, row, row, full((3 * H, H)), full((1, H))],
        out_specs=[row, full((8, H))],
        out_shape=[jax.ShapeDtypeStruct((E, H), jnp.bfloat16), st_shape],
        scratch_shapes=[pltpu.VMEM((8, H), jnp.float32)],
        interpret=_INTERPRET,
    )(xsrc, es, xdst, fi_W1, vec(fi_b1))

    (st2,) = pl.pallas_call(
        functools.partial(_passB_body, float(E)),
        grid=(nb,),
        in_specs=[row, full((8, H)), full((H, H)), full((1, H)),
                  full((1, H)), full((1, H))],
        out_specs=[full((8, H))],
        out_shape=[st_shape],
        scratch_shapes=[pltpu.VMEM((8, H), jnp.float32)],
        interpret=_INTERPRET,
    )(a1, st1, fi_W2, vec(fi_b2), vec(fi_g1), vec(fi_be1))

    m_st = pl.pallas_call(
        functools.partial(_passC_body, float(E)),
        grid=(nb,),
        in_specs=[row, row, row1, full((8, H)), full((8, H)),
                  full((H, H)), full((1, H)), full((1, H)), full((1, H)),
                  full((H, H)), full((1, H)), full((1, H)), full((1, H)),
                  full((H, H)), full((1, H))],
        out_specs=row,
        out_shape=jax.ShapeDtypeStruct((E, H), jnp.float32),
        interpret=_INTERPRET,
    )(a1, xsrc, nd, st1, st2,
      fi_W2, vec(fi_b2), vec(fi_g1), vec(fi_be1),
      fi_W3, vec(fi_b3), vec(fi_g2), vec(fi_be2),
      Wm, vec(bm))
    return m_st


def _node_body(x_ref, inc_ref, Wm_ref, bm_ref, imp_ref,
               W1_ref, b1_ref, g1_ref, be1_ref, W2_ref, b2_ref, out_ref):
    x = x_ref[...]
    n = x.shape[0]
    inc = inc_ref[0] + inc_ref[1]
    mt = imp_ref[...] * (jnp.dot(x, Wm_ref[...], preferred_element_type=jnp.float32)
                         + bm_ref[0:1, :])
    u = mt + inc
    t = _leaky(jnp.dot(u, W1_ref[...], preferred_element_type=jnp.float32)
               + b1_ref[0:1, :])
    m = jnp.sum(t, axis=0, keepdims=True) / n
    v = jnp.sum(t * t, axis=0, keepdims=True) / n - m * m
    t = (t - m) * lax.rsqrt(v + _EPS) * g1_ref[...] + be1_ref[...]
    out_ref[...] = (jnp.dot(t, W2_ref[...], preferred_element_type=jnp.float32)
                    + b2_ref[0:1, :] + x)


def _node_finale(x, parts, Wm, bm, imp_mask, fu_W1, fu_b1, fu_g1, fu_be1,
                 fu_W2, fu_b2):
    N, H = x.shape
    vec = lambda a: a.reshape(1, H)
    return pl.pallas_call(
        _node_body,
        out_shape=jax.ShapeDtypeStruct((N, H), jnp.float32),
        interpret=_INTERPRET,
    )(x, parts, Wm, vec(bm), imp_mask, fu_W1, vec(fu_b1), vec(fu_g1),
      vec(fu_be1), fu_W2, vec(fu_b2))


def kernel(x, edge_index, norm_distance, init_edge_states, Wm, bm, imp_mask,
           fi_W1, fi_b1, fi_g1, fi_be1, fi_W2, fi_b2, fi_g2, fi_be2,
           fi_W3, fi_b3, fu_W1, fu_b1, fu_g1, fu_be1, fu_W2, fu_b2):
    N, H = x.shape
    E = edge_index.shape[1]
    src = edge_index[0]
    dst = edge_index[1]
    dec = _decay(norm_distance)

    xsrc, xdst = _sc_gather(x, src, dst)
    m_st = _edge_passes(xsrc, xdst, init_edge_states, dec,
                        fi_W1, fi_b1, fi_g1, fi_be1,
                        fi_W2, fi_b2, fi_g2, fi_be2,
                        fi_W3, fi_b3, Wm, bm)
    parts = _sc_scatter(m_st, dst, N)
    return _node_finale(x, parts, Wm, bm, imp_mask,
                        fu_W1, fu_b1, fu_g1, fu_be1, fu_W2, fu_b2)


# speedup vs baseline: 3.0992x; 1.1732x over previous
import functools

import jax
import jax.numpy as jnp
from jax import lax
from jax.experimental import pallas as pl
from jax.experimental.pallas import tpu as pltpu
from jax.experimental.pallas import tpu_sc as plsc

_EPS = 1e-5
_INTERPRET = False

_NC = 2
_NS = 16
_CK = 128


def _leaky(v):
    return jnp.where(v >= 0, v, 0.01 * v)


def _gather_body(src_hbm, dst_hbm, x_hbm, xsrc_hbm, xdst_hbm,
                 ia0, ia1, ib0, ib1, ra0, ra1, rb0, rb1,
                 sia0, sia1, sib0, sib1, sga0, sga1, sgb0, sgb1):
    E = src_hbm.shape[0]
    nchunks = E // _CK
    c = lax.axis_index("c")
    s = lax.axis_index("s")
    wid = s * _NC + c
    nw = _NC * _NS
    iters = pl.cdiv(nchunks, nw)
    IA, IB = (ia0, ia1), (ib0, ib1)
    RA, RB = (ra0, ra1), (rb0, rb1)
    SIA, SIB = (sia0, sia1), (sib0, sib1)
    SGA, SGB = (sga0, sga1), (sgb0, sgb1)

    def chunk_of(k):
        return k * nw + wid

    def issue_idx(k, b):
        base = chunk_of(k) * _CK
        pltpu.async_copy(src_hbm.at[pl.ds(base, _CK)], IA[b], SIA[b])
        pltpu.async_copy(dst_hbm.at[pl.ds(base, _CK)], IB[b], SIB[b])

    def wait_idx(b):
        pltpu.make_async_copy(src_hbm.at[pl.ds(0, _CK)], IA[b], SIA[b]).wait()
        pltpu.make_async_copy(dst_hbm.at[pl.ds(0, _CK)], IB[b], SIB[b]).wait()

    def issue_gather(b):
        pltpu.async_copy(x_hbm.at[IA[b]], RA[b], SGA[b])
        pltpu.async_copy(x_hbm.at[IB[b]], RB[b], SGB[b])

    def wait_gather(b):
        pltpu.make_async_copy(x_hbm.at[pl.ds(0, _CK)], RA[b], SGA[b]).wait()
        pltpu.make_async_copy(x_hbm.at[pl.ds(0, _CK)], RB[b], SGB[b]).wait()

    @pl.when(chunk_of(0) < nchunks)
    def _():
        issue_idx(0, 0)

    @pl.when(chunk_of(1) < nchunks)
    def _():
        issue_idx(1, 1)

    @pl.when(chunk_of(0) < nchunks)
    def _():
        wait_idx(0)
        issue_gather(0)

    def body(q, _):
        for b in (0, 1):
            k = 2 * q + b

            @pl.when(chunk_of(k) < nchunks)
            def _():
                @pl.when(chunk_of(k + 1) < nchunks)
                def _():
                    wait_idx(1 - b)
                    issue_gather(1 - b)

                wait_gather(b)
                base = chunk_of(k) * _CK
                pltpu.sync_copy(RA[b], xsrc_hbm.at[pl.ds(base, _CK)])
                pltpu.sync_copy(RB[b], xdst_hbm.at[pl.ds(base, _CK)])

                @pl.when(chunk_of(k + 2) < nchunks)
                def _():
                    issue_idx(k + 2, b)

        return 0

    lax.fori_loop(0, pl.cdiv(iters, 2), body, 0)


def _sc_gather(x, src, dst):
    E = src.shape[0]
    H = x.shape[1]
    mesh = plsc.VectorSubcoreMesh(core_axis_name="c", subcore_axis_name="s")
    f = pl.kernel(
        _gather_body,
        out_type=(
            jax.ShapeDtypeStruct((E, H), jnp.float32),
            jax.ShapeDtypeStruct((E, H), jnp.float32),
        ),
        mesh=mesh,
        scratch_types=(
            [pltpu.VMEM((_CK,), jnp.int32)] * 4
            + [pltpu.VMEM((_CK, H), jnp.float32)] * 4
            + [pltpu.SemaphoreType.DMA] * 8
        ),
    )
    return f(src, dst, x)


def _scatter_body(dst_hbm, mst_hbm, out_hbm, i0, i1, r0, r1, acc,
                  si0, si1, sr0, sr1):
    E = dst_hbm.shape[0]
    N = acc.shape[0]
    c = lax.axis_index("c")
    s = lax.axis_index("s")
    e_half = E // _NC
    nchunks = e_half // _CK
    iters = pl.cdiv(nchunks, _NS)
    I, R = (i0, i1), (r0, r1)
    SI, SR = (si0, si1), (sr0, sr1)

    ntiles_z = 10
    rows_per_tile = N // ntiles_z
    zrows = r0.shape[0]

    def zero_buf(r, _):
        for j in range(8):
            r0[r, pl.ds(j * 16, 16)] = jnp.zeros((16,), jnp.float32)
        return 0

    lax.fori_loop(0, zrows, zero_buf, 0)

    nz = rows_per_tile // zrows
    rem = rows_per_tile - nz * zrows

    @pl.when(s < ntiles_z)
    def _():
        def zero_acc(q, _):
            pltpu.sync_copy(r0,
                            acc.at[pl.ds(s * rows_per_tile + q * zrows, zrows)])
            return 0

        lax.fori_loop(0, nz, zero_acc, 0)
        if rem:
            pltpu.sync_copy(r0.at[pl.ds(0, rem)],
                            acc.at[pl.ds(s * rows_per_tile + nz * zrows, rem)])

    plsc.subcore_barrier()

    def chunk_of(k):
        return k * _NS + s

    def issue_load(k, b):
        base = c * e_half + chunk_of(k) * _CK
        pltpu.async_copy(dst_hbm.at[pl.ds(base, _CK)], I[b], SI[b])
        pltpu.async_copy(mst_hbm.at[pl.ds(base, _CK)], R[b], SR[b])

    def wait_load(b):
        pltpu.make_async_copy(dst_hbm.at[pl.ds(0, _CK)], I[b], SI[b]).wait()
        pltpu.make_async_copy(mst_hbm.at[pl.ds(0, _CK)], R[b], SR[b]).wait()

    @pl.when(chunk_of(0) < nchunks)
    def _():
        issue_load(0, 0)

    def body(q, _):
        for b in (0, 1):
            k = 2 * q + b

            @pl.when(chunk_of(k) < nchunks)
            def _():
                @pl.when(chunk_of(k + 1) < nchunks)
                def _():
                    issue_load(k + 1, 1 - b)

                wait_load(b)
                pltpu.sync_copy(R[b], acc.at[I[b]], add=True)

        return 0

    lax.fori_loop(0, pl.cdiv(iters, 2), body, 0)

    plsc.subcore_barrier()

    @pl.when(s < ntiles_z)
    def _():
        pltpu.sync_copy(acc.at[pl.ds(s * rows_per_tile, rows_per_tile)],
                        out_hbm.at[c, pl.ds(s * rows_per_tile, rows_per_tile)])


def _sc_scatter(m_st, dst, N):
    E, H = m_st.shape
    mesh = plsc.VectorSubcoreMesh(core_axis_name="c", subcore_axis_name="s")
    f = pl.kernel(
        _scatter_body,
        out_type=jax.ShapeDtypeStruct((_NC, N, H), jnp.float32),
        mesh=mesh,
        scratch_types=(
            [pltpu.VMEM((_CK,), jnp.int32)] * 2
            + [pltpu.VMEM((_CK, H), jnp.float32)] * 2
            + [pltpu.VMEM_SHARED((N, H), jnp.float32)]
            + [pltpu.SemaphoreType.DMA] * 4
        ),
    )
    return f(dst, m_st)


def _decay_body(nd_ref, out_ref):
    t = (jnp.pi / 2.0) * nd_ref[...]
    t2 = t * t
    c = 1.0 / 479001600.0
    for k in (-1.0 / 3628800.0, 1.0 / 40320.0, -1.0 / 720.0, 1.0 / 24.0,
              -0.5, 1.0):
        c = c * t2 + k
    out_ref[...] = c


def _decay(norm_distance):
    E = norm_distance.shape[0]
    nd2 = norm_distance.reshape(E // 128, 128)
    out = pl.pallas_call(
        _decay_body,
        out_shape=jax.ShapeDtypeStruct(nd2.shape, jnp.float32),
        interpret=_INTERPRET,
    )(nd2)
    return out.reshape(E, 1)


_BE = 2560


def _passA_body(xsrc_ref, es_ref, xdst_ref, W1_ref, b1_ref,
                a1_ref, st1_ref, acc):
    i = pl.program_id(0)

    @pl.when(i == 0)
    def _():
        acc[...] = jnp.zeros_like(acc)

    h = jnp.concatenate([xsrc_ref[...], es_ref[...], xdst_ref[...]],
                        axis=1).astype(jnp.bfloat16)
    p = jnp.dot(h, W1_ref[...].astype(jnp.bfloat16),
                preferred_element_type=jnp.float32) + b1_ref[0:1, :]
    a = _leaky(p)
    a1_ref[...] = a.astype(jnp.bfloat16)
    acc[0:1, :] += jnp.sum(a, axis=0, keepdims=True)
    acc[1:2, :] += jnp.sum(a * a, axis=0, keepdims=True)

    @pl.when(i == pl.num_programs(0) - 1)
    def _():
        st1_ref[...] = acc[...]


def _bn_scale(st_ref, g_ref, be_ref, count):
    m = st_ref[0:1, :] / count
    v = st_ref[1:2, :] / count - m * m
    s = g_ref[...] * lax.rsqrt(v + _EPS)
    t = be_ref[...] - m * s
    return s, t


def _folded_layer2(a1_bf, st1_ref, W2_ref, b2_ref, g1_ref, be1_ref, E):
    s1, t1 = _bn_scale(st1_ref, g1_ref, be1_ref, E)
    W2f = (jnp.transpose(s1) * W2_ref[...]).astype(jnp.bfloat16)
    bias = (jnp.dot(t1, W2_ref[...], preferred_element_type=jnp.float32)
            + b2_ref[0:1, :])
    return _leaky(jnp.dot(a1_bf, W2f, preferred_element_type=jnp.float32)
                  + bias)


def _passB_body(E, a1_ref, st1_ref, W2_ref, b2_ref, g1_ref, be1_ref,
                st2_ref, acc):
    i = pl.program_id(0)

    @pl.when(i == 0)
    def _():
        acc[...] = jnp.zeros_like(acc)

    a2 = _folded_layer2(a1_ref[...], st1_ref, W2_ref, b2_ref, g1_ref,
                        be1_ref, E)
    acc[0:1, :] += jnp.sum(a2, axis=0, keepdims=True)
    acc[1:2, :] += jnp.sum(a2 * a2, axis=0, keepdims=True)

    @pl.when(i == pl.num_programs(0) - 1)
    def _():
        st2_ref[...] = acc[...]


def _passC_body(E, a1_ref, xsrc_ref, nd_ref, st1_ref, st2_ref,
                W2_ref, b2_ref, g1_ref, be1_ref,
                W3_ref, b3_ref, g2_ref, be2_ref,
                Wm_ref, bm_ref, out_ref):
    a2 = _folded_layer2(a1_ref[...], st1_ref, W2_ref, b2_ref, g1_ref,
                        be1_ref, E)
    s2, t2 = _bn_scale(st2_ref, g2_ref, be2_ref, E)
    W3f = (jnp.transpose(s2) * W3_ref[...]).astype(jnp.bfloat16)
    bias3 = (jnp.dot(t2, W3_ref[...], preferred_element_type=jnp.float32)
             + b3_ref[0:1, :])
    h3 = jnp.dot(a2.astype(jnp.bfloat16), W3f,
                 preferred_element_type=jnp.float32) + bias3
    G = jnp.dot(xsrc_ref[...].astype(jnp.bfloat16),
                Wm_ref[...].astype(jnp.bfloat16),
                preferred_element_type=jnp.float32) + bm_ref[0:1, :]
    out_ref[...] = nd_ref[...] * h3 * G


def _row_block(i):
    return (i, 0)


def _pinned(i):
    return (0, 0)


def _edge_passes(xsrc, xdst, es, nd, fi_W1, fi_b1, fi_g1, fi_be1,
                 fi_W2, fi_b2, fi_g2, fi_be2, fi_W3, fi_b3, Wm, bm):
    E, H = xsrc.shape
    nb = E // _BE
    row = pl.BlockSpec((_BE, H), _row_block)
    row1 = pl.BlockSpec((_BE, 1), _row_block)
    full = lambda shape: pl.BlockSpec(shape, _pinned)
    st_shape = jax.ShapeDtypeStruct((8, H), jnp.float32)
    vec = lambda a: a.reshape(1, H)

    a1, st1 = pl.pallas_call(
        _passA_body,
        grid=(nb,),
        in_specs=[row, row, row, full((3 * H, H)), full((1, H))],
        out_specs=[row, full((8, H))],
        out_shape=[jax.ShapeDtypeStruct((E, H), jnp.bfloat16), st_shape],
        scratch_shapes=[pltpu.VMEM((8, H), jnp.float32)],
        interpret=_INTERPRET,
    )(xsrc, es, xdst, fi_W1, vec(fi_b1))

    (st2,) = pl.pallas_call(
        functools.partial(_passB_body, float(E)),
        grid=(nb,),
        in_specs=[row, full((8, H)), full((H, H)), full((1, H)),
                  full((1, H)), full((1, H))],
        out_specs=[full((8, H))],
        out_shape=[st_shape],
        scratch_shapes=[pltpu.VMEM((8, H), jnp.float32)],
        interpret=_INTERPRET,
    )(a1, st1, fi_W2, vec(fi_b2), vec(fi_g1), vec(fi_be1))

    m_st = pl.pallas_call(
        functools.partial(_passC_body, float(E)),
        grid=(nb,),
        in_specs=[row, row, row1, full((8, H)), full((8, H)),
                  full((H, H)), full((1, H)), full((1, H)), full((1, H)),
                  full((H, H)), full((1, H)), full((1, H)), full((1, H)),
                  full((H, H)), full((1, H))],
        out_specs=row,
        out_shape=jax.ShapeDtypeStruct((E, H), jnp.float32),
        interpret=_INTERPRET,
    )(a1, xsrc, nd, st1, st2,
      fi_W2, vec(fi_b2), vec(fi_g1), vec(fi_be1),
      fi_W3, vec(fi_b3), vec(fi_g2), vec(fi_be2),
      Wm, vec(bm))
    return m_st


def _node_body(x_ref, inc_ref, Wm_ref, bm_ref, imp_ref,
               W1_ref, b1_ref, g1_ref, be1_ref, W2_ref, b2_ref, out_ref):
    x = x_ref[...]
    n = x.shape[0]
    inc = inc_ref[0] + inc_ref[1]
    mt = imp_ref[...] * (jnp.dot(x, Wm_ref[...], preferred_element_type=jnp.float32)
                         + bm_ref[0:1, :])
    u = mt + inc
    t = _leaky(jnp.dot(u, W1_ref[...], preferred_element_type=jnp.float32)
               + b1_ref[0:1, :])
    m = jnp.sum(t, axis=0, keepdims=True) / n
    v = jnp.sum(t * t, axis=0, keepdims=True) / n - m * m
    t = (t - m) * lax.rsqrt(v + _EPS) * g1_ref[...] + be1_ref[...]
    out_ref[...] = (jnp.dot(t, W2_ref[...], preferred_element_type=jnp.float32)
                    + b2_ref[0:1, :] + x)


def _node_finale(x, parts, Wm, bm, imp_mask, fu_W1, fu_b1, fu_g1, fu_be1,
                 fu_W2, fu_b2):
    N, H = x.shape
    vec = lambda a: a.reshape(1, H)
    return pl.pallas_call(
        _node_body,
        out_shape=jax.ShapeDtypeStruct((N, H), jnp.float32),
        interpret=_INTERPRET,
    )(x, parts, Wm, vec(bm), imp_mask, fu_W1, vec(fu_b1), vec(fu_g1),
      vec(fu_be1), fu_W2, vec(fu_b2))


def kernel(x, edge_index, norm_distance, init_edge_states, Wm, bm, imp_mask,
           fi_W1, fi_b1, fi_g1, fi_be1, fi_W2, fi_b2, fi_g2, fi_be2,
           fi_W3, fi_b3, fu_W1, fu_b1, fu_g1, fu_be1, fu_W2, fu_b2):
    N, H = x.shape
    E = edge_index.shape[1]
    src = edge_index[0]
    dst = edge_index[1]
    dec = _decay(norm_distance)

    xsrc, xdst = _sc_gather(x, src, dst)
    m_st = _edge_passes(xsrc, xdst, init_edge_states, dec,
                        fi_W1, fi_b1, fi_g1, fi_be1,
                        fi_W2, fi_b2, fi_g2, fi_be2,
                        fi_W3, fi_b3, Wm, bm)
    parts = _sc_scatter(m_st, dst, N)
    return _node_finale(x, parts, Wm, bm, imp_mask,
                        fu_W1, fu_b1, fu_g1, fu_be1, fu_W2, fu_b2)


# speedup vs baseline: 3.3790x; 1.0903x over previous
import functools

import jax
import jax.numpy as jnp
from jax import lax
from jax.experimental import pallas as pl
from jax.experimental.pallas import tpu as pltpu
from jax.experimental.pallas import tpu_sc as plsc

_EPS = 1e-5
_INTERPRET = False

_NC = 2
_NS = 16
_CK = 128


def _leaky(v):
    return jnp.where(v >= 0, v, 0.01 * v)


def _gather_body(src_hbm, dst_hbm, x_hbm, xsrc_hbm, xdst_hbm,
                 ia0, ia1, ib0, ib1, ra0, ra1, rb0, rb1,
                 sia0, sia1, sib0, sib1, sga0, sga1, sgb0, sgb1):
    E = src_hbm.shape[0]
    nchunks = E // _CK
    c = lax.axis_index("c")
    s = lax.axis_index("s")
    wid = s * _NC + c
    nw = _NC * _NS
    iters = pl.cdiv(nchunks, nw)
    IA, IB = (ia0, ia1), (ib0, ib1)
    RA, RB = (ra0, ra1), (rb0, rb1)
    SIA, SIB = (sia0, sia1), (sib0, sib1)
    SGA, SGB = (sga0, sga1), (sgb0, sgb1)

    def chunk_of(k):
        return k * nw + wid

    def issue_idx(k, b):
        base = chunk_of(k) * _CK
        pltpu.async_copy(src_hbm.at[pl.ds(base, _CK)], IA[b], SIA[b])
        pltpu.async_copy(dst_hbm.at[pl.ds(base, _CK)], IB[b], SIB[b])

    def wait_idx(b):
        pltpu.make_async_copy(src_hbm.at[pl.ds(0, _CK)], IA[b], SIA[b]).wait()
        pltpu.make_async_copy(dst_hbm.at[pl.ds(0, _CK)], IB[b], SIB[b]).wait()

    def issue_gather(b):
        pltpu.async_copy(x_hbm.at[IA[b]], RA[b], SGA[b])
        pltpu.async_copy(x_hbm.at[IB[b]], RB[b], SGB[b])

    def wait_gather(b):
        pltpu.make_async_copy(x_hbm.at[pl.ds(0, _CK)], RA[b], SGA[b]).wait()
        pltpu.make_async_copy(x_hbm.at[pl.ds(0, _CK)], RB[b], SGB[b]).wait()

    @pl.when(chunk_of(0) < nchunks)
    def _():
        issue_idx(0, 0)

    @pl.when(chunk_of(1) < nchunks)
    def _():
        issue_idx(1, 1)

    @pl.when(chunk_of(0) < nchunks)
    def _():
        wait_idx(0)
        issue_gather(0)

    def body(q, _):
        for b in (0, 1):
            k = 2 * q + b

            @pl.when(chunk_of(k) < nchunks)
            def _():
                @pl.when(chunk_of(k + 1) < nchunks)
                def _():
                    wait_idx(1 - b)
                    issue_gather(1 - b)

                wait_gather(b)
                base = chunk_of(k) * _CK
                pltpu.sync_copy(RA[b], xsrc_hbm.at[pl.ds(base, _CK)])
                pltpu.sync_copy(RB[b], xdst_hbm.at[pl.ds(base, _CK)])

                @pl.when(chunk_of(k + 2) < nchunks)
                def _():
                    issue_idx(k + 2, b)

        return 0

    lax.fori_loop(0, pl.cdiv(iters, 2), body, 0)


def _sc_gather(x, src, dst):
    E = src.shape[0]
    H = x.shape[1]
    mesh = plsc.VectorSubcoreMesh(core_axis_name="c", subcore_axis_name="s")
    f = pl.kernel(
        _gather_body,
        out_type=(
            jax.ShapeDtypeStruct((E, H), jnp.float32),
            jax.ShapeDtypeStruct((E, H), jnp.float32),
        ),
        mesh=mesh,
        scratch_types=(
            [pltpu.VMEM((_CK,), jnp.int32)] * 4
            + [pltpu.VMEM((_CK, H), jnp.float32)] * 4
            + [pltpu.SemaphoreType.DMA] * 8
        ),
    )
    return f(src, dst, x)


def _scatter_body(dst_hbm, mst_hbm, out_hbm, i0, i1, r0, r1, acc,
                  si0, si1, sr0, sr1):
    E = dst_hbm.shape[0]
    N = acc.shape[0]
    c = lax.axis_index("c")
    s = lax.axis_index("s")
    e_half = E // _NC
    nchunks = e_half // _CK
    iters = pl.cdiv(nchunks, _NS)
    I, R = (i0, i1), (r0, r1)
    SI, SR = (si0, si1), (sr0, sr1)

    ntiles_z = 10
    rows_per_tile = N // ntiles_z
    zrows = r0.shape[0]

    def zero_buf(r, _):
        for j in range(8):
            r0[r, pl.ds(j * 16, 16)] = jnp.zeros((16,), jnp.float32)
        return 0

    lax.fori_loop(0, zrows, zero_buf, 0)

    nz = rows_per_tile // zrows
    rem = rows_per_tile - nz * zrows

    @pl.when(s < ntiles_z)
    def _():
        def zero_acc(q, _):
            pltpu.sync_copy(r0,
                            acc.at[pl.ds(s * rows_per_tile + q * zrows, zrows)])
            return 0

        lax.fori_loop(0, nz, zero_acc, 0)
        if rem:
            pltpu.sync_copy(r0.at[pl.ds(0, rem)],
                            acc.at[pl.ds(s * rows_per_tile + nz * zrows, rem)])

    plsc.subcore_barrier()

    def chunk_of(k):
        return k * _NS + s

    def issue_load(k, b):
        base = c * e_half + chunk_of(k) * _CK
        pltpu.async_copy(dst_hbm.at[pl.ds(base, _CK)], I[b], SI[b])
        pltpu.async_copy(mst_hbm.at[pl.ds(base, _CK)], R[b], SR[b])

    def wait_load(b):
        pltpu.make_async_copy(dst_hbm.at[pl.ds(0, _CK)], I[b], SI[b]).wait()
        pltpu.make_async_copy(mst_hbm.at[pl.ds(0, _CK)], R[b], SR[b]).wait()

    @pl.when(chunk_of(0) < nchunks)
    def _():
        issue_load(0, 0)

    def body(q, _):
        for b in (0, 1):
            k = 2 * q + b

            @pl.when(chunk_of(k) < nchunks)
            def _():
                @pl.when(chunk_of(k + 1) < nchunks)
                def _():
                    issue_load(k + 1, 1 - b)

                wait_load(b)
                pltpu.sync_copy(R[b], acc.at[I[b]], add=True)

        return 0

    lax.fori_loop(0, pl.cdiv(iters, 2), body, 0)

    plsc.subcore_barrier()

    @pl.when(s < ntiles_z)
    def _():
        pltpu.sync_copy(acc.at[pl.ds(s * rows_per_tile, rows_per_tile)],
                        out_hbm.at[c, pl.ds(s * rows_per_tile, rows_per_tile)])


def _sc_scatter(m_st, dst, N):
    E, H = m_st.shape
    mesh = plsc.VectorSubcoreMesh(core_axis_name="c", subcore_axis_name="s")
    f = pl.kernel(
        _scatter_body,
        out_type=jax.ShapeDtypeStruct((_NC, N, H), jnp.float32),
        mesh=mesh,
        scratch_types=(
            [pltpu.VMEM((_CK,), jnp.int32)] * 2
            + [pltpu.VMEM((_CK, H), jnp.float32)] * 2
            + [pltpu.VMEM_SHARED((N, H), jnp.float32)]
            + [pltpu.SemaphoreType.DMA] * 4
        ),
    )
    return f(dst, m_st)


def _decay_body(nd_ref, out_ref):
    t = (jnp.pi / 2.0) * nd_ref[...]
    t2 = t * t
    c = 1.0 / 479001600.0
    for k in (-1.0 / 3628800.0, 1.0 / 40320.0, -1.0 / 720.0, 1.0 / 24.0,
              -0.5, 1.0):
        c = c * t2 + k
    out_ref[...] = c


def _decay(norm_distance):
    E = norm_distance.shape[0]
    nd2 = norm_distance.reshape(E // 128, 128)
    return pl.pallas_call(
        _decay_body,
        out_shape=jax.ShapeDtypeStruct(nd2.shape, jnp.float32),
        interpret=_INTERPRET,
    )(nd2)


_BE = 2560


def _passA_body(xsrc_ref, es_ref, xdst_ref, W1_ref, b1_ref,
                a1_ref, st1_ref, acc):
    i = pl.program_id(0)

    @pl.when(i == 0)
    def _():
        acc[...] = jnp.zeros_like(acc)

    h = jnp.concatenate([xsrc_ref[...], es_ref[...], xdst_ref[...]],
                        axis=1).astype(jnp.bfloat16)
    p = jnp.dot(h, W1_ref[...].astype(jnp.bfloat16),
                preferred_element_type=jnp.float32) + b1_ref[0:1, :]
    a = _leaky(p)
    a1_ref[...] = a.astype(jnp.bfloat16)
    acc[0:1, :] += jnp.sum(a, axis=0, keepdims=True)
    acc[1:2, :] += jnp.sum(a * a, axis=0, keepdims=True)

    @pl.when(i == pl.num_programs(0) - 1)
    def _():
        st1_ref[...] = acc[...]


def _bn_scale(st_ref, g_ref, be_ref, count):
    m = st_ref[0:1, :] / count
    v = st_ref[1:2, :] / count - m * m
    s = g_ref[...] * lax.rsqrt(v + _EPS)
    t = be_ref[...] - m * s
    return s, t


def _folded_layer2(a1_bf, st1_ref, W2_ref, b2_ref, g1_ref, be1_ref, E):
    s1, t1 = _bn_scale(st1_ref, g1_ref, be1_ref, E)
    W2f = (jnp.transpose(s1) * W2_ref[...]).astype(jnp.bfloat16)
    bias = (jnp.dot(t1, W2_ref[...], preferred_element_type=jnp.float32)
            + b2_ref[0:1, :])
    return _leaky(jnp.dot(a1_bf, W2f, preferred_element_type=jnp.float32)
                  + bias)


def _passB_body(E, a1_ref, st1_ref, W2_ref, b2_ref, g1_ref, be1_ref,
                st2_ref, acc):
    i = pl.program_id(0)

    @pl.when(i == 0)
    def _():
        acc[...] = jnp.zeros_like(acc)

    a2 = _folded_layer2(a1_ref[...], st1_ref, W2_ref, b2_ref, g1_ref,
                        be1_ref, E)
    acc[0:1, :] += jnp.sum(a2, axis=0, keepdims=True)
    acc[1:2, :] += jnp.sum(a2 * a2, axis=0, keepdims=True)

    @pl.when(i == pl.num_programs(0) - 1)
    def _():
        st2_ref[...] = acc[...]


def _passC_body(E, a1_ref, xsrc_ref, nd_ref, st1_ref, st2_ref,
                W2_ref, b2_ref, g1_ref, be1_ref,
                W3_ref, b3_ref, g2_ref, be2_ref,
                Wm_ref, bm_ref, out_ref):
    a2 = _folded_layer2(a1_ref[...], st1_ref, W2_ref, b2_ref, g1_ref,
                        be1_ref, E)
    s2, t2 = _bn_scale(st2_ref, g2_ref, be2_ref, E)
    W3f = (jnp.transpose(s2) * W3_ref[...]).astype(jnp.bfloat16)
    bias3 = (jnp.dot(t2, W3_ref[...], preferred_element_type=jnp.float32)
             + b3_ref[0:1, :])
    h3 = jnp.dot(a2.astype(jnp.bfloat16), W3f,
                 preferred_element_type=jnp.float32) + bias3
    G = jnp.dot(xsrc_ref[...].astype(jnp.bfloat16),
                Wm_ref[...].astype(jnp.bfloat16),
                preferred_element_type=jnp.float32) + bm_ref[0:1, :]
    i = pl.program_id(0)
    nb = h3.shape[0]
    dec = jnp.reshape(nd_ref[pl.ds(i * nb, nb)], (nb, 1))
    out_ref[...] = dec * h3 * G


def _row_block(i):
    return (i, 0)


def _pinned(i):
    return (0, 0)


def _edge_passes(xsrc, xdst, es, nd, fi_W1, fi_b1, fi_g1, fi_be1,
                 fi_W2, fi_b2, fi_g2, fi_be2, fi_W3, fi_b3, Wm, bm):
    E, H = xsrc.shape
    nb = E // _BE
    row = pl.BlockSpec((_BE, H), _row_block)
    rowd = pl.BlockSpec((E,), lambda i: (0,))
    full = lambda shape: pl.BlockSpec(shape, _pinned)
    st_shape = jax.ShapeDtypeStruct((8, H), jnp.float32)
    vec = lambda a: a.reshape(1, H)

    a1, st1 = pl.pallas_call(
        _passA_body,
        grid=(nb,),
        in_specs=[row, row, row, full((3 * H, H)), full((1, H))],
        out_specs=[row, full((8, H))],
        out_shape=[jax.ShapeDtypeStruct((E, H), jnp.bfloat16), st_shape],
        scratch_shapes=[pltpu.VMEM((8, H), jnp.float32)],
        interpret=_INTERPRET,
    )(xsrc, es, xdst, fi_W1, vec(fi_b1))

    (st2,) = pl.pallas_call(
        functools.partial(_passB_body, float(E)),
        grid=(nb,),
        in_specs=[row, full((8, H)), full((H, H)), full((1, H)),
                  full((1, H)), full((1, H))],
        out_specs=[full((8, H))],
        out_shape=[st_shape],
        scratch_shapes=[pltpu.VMEM((8, H), jnp.float32)],
        interpret=_INTERPRET,
    )(a1, st1, fi_W2, vec(fi_b2), vec(fi_g1), vec(fi_be1))

    nd3 = nd.reshape(E)
    m_st = pl.pallas_call(
        functools.partial(_passC_body, float(E)),
        grid=(nb,),
        in_specs=[row, row, rowd, full((8, H)), full((8, H)),
                  full((H, H)), full((1, H)), full((1, H)), full((1, H)),
                  full((H, H)), full((1, H)), full((1, H)), full((1, H)),
                  full((H, H)), full((1, H))],
        out_specs=row,
        out_shape=jax.ShapeDtypeStruct((E, H), jnp.float32),
        interpret=_INTERPRET,
    )(a1, xsrc, nd3, st1, st2,
      fi_W2, vec(fi_b2), vec(fi_g1), vec(fi_be1),
      fi_W3, vec(fi_b3), vec(fi_g2), vec(fi_be2),
      Wm, vec(bm))
    return m_st


def _node_body(x_ref, inc_ref, Wm_ref, bm_ref, imp_ref,
               W1_ref, b1_ref, g1_ref, be1_ref, W2_ref, b2_ref, out_ref):
    x = x_ref[...]
    n = x.shape[0]
    inc = inc_ref[0] + inc_ref[1]
    mt = imp_ref[...] * (jnp.dot(x, Wm_ref[...], preferred_element_type=jnp.float32)
                         + bm_ref[0:1, :])
    u = mt + inc
    t = _leaky(jnp.dot(u, W1_ref[...], preferred_element_type=jnp.float32)
               + b1_ref[0:1, :])
    m = jnp.sum(t, axis=0, keepdims=True) / n
    v = jnp.sum(t * t, axis=0, keepdims=True) / n - m * m
    t = (t - m) * lax.rsqrt(v + _EPS) * g1_ref[...] + be1_ref[...]
    out_ref[...] = (jnp.dot(t, W2_ref[...], preferred_element_type=jnp.float32)
                    + b2_ref[0:1, :] + x)


def _node_finale(x, parts, Wm, bm, imp_mask, fu_W1, fu_b1, fu_g1, fu_be1,
                 fu_W2, fu_b2):
    N, H = x.shape
    vec = lambda a: a.reshape(1, H)
    return pl.pallas_call(
        _node_body,
        out_shape=jax.ShapeDtypeStruct((N, H), jnp.float32),
        interpret=_INTERPRET,
    )(x, parts, Wm, vec(bm), imp_mask, fu_W1, vec(fu_b1), vec(fu_g1),
      vec(fu_be1), fu_W2, vec(fu_b2))


def kernel(x, edge_index, norm_distance, init_edge_states, Wm, bm, imp_mask,
           fi_W1, fi_b1, fi_g1, fi_be1, fi_W2, fi_b2, fi_g2, fi_be2,
           fi_W3, fi_b3, fu_W1, fu_b1, fu_g1, fu_be1, fu_W2, fu_b2):
    N, H = x.shape
    E = edge_index.shape[1]
    src = edge_index[0]
    dst = edge_index[1]
    dec = _decay(norm_distance)

    xsrc, xdst = _sc_gather(x, src, dst)
    m_st = _edge_passes(xsrc, xdst, init_edge_states, dec,
                        fi_W1, fi_b1, fi_g1, fi_be1,
                        fi_W2, fi_b2, fi_g2, fi_be2,
                        fi_W3, fi_b3, Wm, bm)
    parts = _sc_scatter(m_st, dst, N)
    return _node_finale(x, parts, Wm, bm, imp_mask,
                        fu_W1, fu_b1, fu_g1, fu_be1, fu_W2, fu_b2)


# speedup vs baseline: 3.3892x; 1.0030x over previous
import functools

import jax
import jax.numpy as jnp
from jax import lax
from jax.experimental import pallas as pl
from jax.experimental.pallas import tpu as pltpu
from jax.experimental.pallas import tpu_sc as plsc

_EPS = 1e-5
_INTERPRET = False

_NC = 2
_NS = 16
_CK = 128


def _leaky(v):
    return jnp.where(v >= 0, v, 0.01 * v)


def _gather_body(src_hbm, dst_hbm, x_hbm, xsrc_hbm, xdst_hbm,
                 ia0, ia1, ib0, ib1, ra0, ra1, rb0, rb1,
                 sia0, sia1, sib0, sib1, sga0, sga1, sgb0, sgb1):
    E = src_hbm.shape[0]
    nchunks = E // _CK
    c = lax.axis_index("c")
    s = lax.axis_index("s")
    wid = s * _NC + c
    nw = _NC * _NS
    iters = pl.cdiv(nchunks, nw)
    IA, IB = (ia0, ia1), (ib0, ib1)
    RA, RB = (ra0, ra1), (rb0, rb1)
    SIA, SIB = (sia0, sia1), (sib0, sib1)
    SGA, SGB = (sga0, sga1), (sgb0, sgb1)

    def chunk_of(k):
        return k * nw + wid

    def issue_idx(k, b):
        base = chunk_of(k) * _CK
        pltpu.async_copy(src_hbm.at[pl.ds(base, _CK)], IA[b], SIA[b])
        pltpu.async_copy(dst_hbm.at[pl.ds(base, _CK)], IB[b], SIB[b])

    def wait_idx(b):
        pltpu.make_async_copy(src_hbm.at[pl.ds(0, _CK)], IA[b], SIA[b]).wait()
        pltpu.make_async_copy(dst_hbm.at[pl.ds(0, _CK)], IB[b], SIB[b]).wait()

    def issue_gather(b):
        pltpu.async_copy(x_hbm.at[IA[b]], RA[b], SGA[b])
        pltpu.async_copy(x_hbm.at[IB[b]], RB[b], SGB[b])

    def wait_gather(b):
        pltpu.make_async_copy(x_hbm.at[pl.ds(0, _CK)], RA[b], SGA[b]).wait()
        pltpu.make_async_copy(x_hbm.at[pl.ds(0, _CK)], RB[b], SGB[b]).wait()

    @pl.when(chunk_of(0) < nchunks)
    def _():
        issue_idx(0, 0)

    @pl.when(chunk_of(1) < nchunks)
    def _():
        issue_idx(1, 1)

    @pl.when(chunk_of(0) < nchunks)
    def _():
        wait_idx(0)
        issue_gather(0)

    def body(q, _):
        for b in (0, 1):
            k = 2 * q + b

            @pl.when(chunk_of(k) < nchunks)
            def _():
                @pl.when(chunk_of(k + 1) < nchunks)
                def _():
                    wait_idx(1 - b)
                    issue_gather(1 - b)

                wait_gather(b)
                base = chunk_of(k) * _CK
                pltpu.sync_copy(RA[b], xsrc_hbm.at[pl.ds(base, _CK)])
                pltpu.sync_copy(RB[b], xdst_hbm.at[pl.ds(base, _CK)])

                @pl.when(chunk_of(k + 2) < nchunks)
                def _():
                    issue_idx(k + 2, b)

        return 0

    lax.fori_loop(0, pl.cdiv(iters, 2), body, 0)


def _sc_gather(x, src, dst):
    E = src.shape[0]
    H = x.shape[1]
    mesh = plsc.VectorSubcoreMesh(core_axis_name="c", subcore_axis_name="s")
    f = pl.kernel(
        _gather_body,
        out_type=(
            jax.ShapeDtypeStruct((E, H), jnp.float32),
            jax.ShapeDtypeStruct((E, H), jnp.float32),
        ),
        mesh=mesh,
        scratch_types=(
            [pltpu.VMEM((_CK,), jnp.int32)] * 4
            + [pltpu.VMEM((_CK, H), jnp.float32)] * 4
            + [pltpu.SemaphoreType.DMA] * 8
        ),
    )
    return f(src, dst, x)


def _scatter_body(dst_hbm, mst_hbm, out_hbm, i0, i1, r0, r1, acc,
                  si0, si1, sr0, sr1):
    E = dst_hbm.shape[0]
    N = acc.shape[0]
    c = lax.axis_index("c")
    s = lax.axis_index("s")
    e_half = E // _NC
    nchunks = e_half // _CK
    iters = pl.cdiv(nchunks, _NS)
    I, R = (i0, i1), (r0, r1)
    SI, SR = (si0, si1), (sr0, sr1)

    ntiles_z = 10
    rows_per_tile = N // ntiles_z
    zrows = r0.shape[0]

    def zero_buf(r, _):
        for j in range(8):
            r0[r, pl.ds(j * 16, 16)] = jnp.zeros((16,), jnp.float32)
        return 0

    lax.fori_loop(0, zrows, zero_buf, 0)

    nz = rows_per_tile // zrows
    rem = rows_per_tile - nz * zrows

    @pl.when(s < ntiles_z)
    def _():
        def zero_acc(q, _):
            pltpu.sync_copy(r0,
                            acc.at[pl.ds(s * rows_per_tile + q * zrows, zrows)])
            return 0

        lax.fori_loop(0, nz, zero_acc, 0)
        if rem:
            pltpu.sync_copy(r0.at[pl.ds(0, rem)],
                            acc.at[pl.ds(s * rows_per_tile + nz * zrows, rem)])

    plsc.subcore_barrier()

    def chunk_of(k):
        return k * _NS + s

    def issue_load(k, b):
        base = c * e_half + chunk_of(k) * _CK
        pltpu.async_copy(dst_hbm.at[pl.ds(base, _CK)], I[b], SI[b])
        pltpu.async_copy(mst_hbm.at[pl.ds(base, _CK)], R[b], SR[b])

    def wait_load(b):
        pltpu.make_async_copy(dst_hbm.at[pl.ds(0, _CK)], I[b], SI[b]).wait()
        pltpu.make_async_copy(mst_hbm.at[pl.ds(0, _CK)], R[b], SR[b]).wait()

    @pl.when(chunk_of(0) < nchunks)
    def _():
        issue_load(0, 0)

    def body(q, _):
        for b in (0, 1):
            k = 2 * q + b

            @pl.when(chunk_of(k) < nchunks)
            def _():
                @pl.when(chunk_of(k + 1) < nchunks)
                def _():
                    issue_load(k + 1, 1 - b)

                wait_load(b)
                pltpu.sync_copy(R[b], acc.at[I[b]], add=True)

        return 0

    lax.fori_loop(0, pl.cdiv(iters, 2), body, 0)

    plsc.subcore_barrier()

    @pl.when(s < ntiles_z)
    def _():
        pltpu.sync_copy(acc.at[pl.ds(s * rows_per_tile, rows_per_tile)],
                        out_hbm.at[c, pl.ds(s * rows_per_tile, rows_per_tile)])


def _sc_scatter(m_st, dst, N):
    E, H = m_st.shape
    mesh = plsc.VectorSubcoreMesh(core_axis_name="c", subcore_axis_name="s")
    f = pl.kernel(
        _scatter_body,
        out_type=jax.ShapeDtypeStruct((_NC, N, H), jnp.float32),
        mesh=mesh,
        scratch_types=(
            [pltpu.VMEM((_CK,), jnp.int32)] * 2
            + [pltpu.VMEM((_CK, H), jnp.float32)] * 2
            + [pltpu.VMEM_SHARED((N, H), jnp.float32)]
            + [pltpu.SemaphoreType.DMA] * 4
        ),
    )
    return f(dst, m_st)


def _decay_poly(nd):
    t = (jnp.pi / 2.0) * nd
    t2 = t * t
    c = 1.0 / 479001600.0
    for k in (-1.0 / 3628800.0, 1.0 / 40320.0, -1.0 / 720.0, 1.0 / 24.0,
              -0.5, 1.0):
        c = c * t2 + k
    return c


_BE = 2560


def _passA_body(xsrc_ref, es_ref, xdst_ref, W1_ref, b1_ref,
                a1_ref, st1_ref, acc):
    i = pl.program_id(0)

    @pl.when(i == 0)
    def _():
        acc[...] = jnp.zeros_like(acc)

    h = jnp.concatenate([xsrc_ref[...], es_ref[...], xdst_ref[...]],
                        axis=1).astype(jnp.bfloat16)
    p = jnp.dot(h, W1_ref[...].astype(jnp.bfloat16),
                preferred_element_type=jnp.float32) + b1_ref[0:1, :]
    a = _leaky(p)
    a1_ref[...] = a.astype(jnp.bfloat16)
    acc[0:1, :] += jnp.sum(a, axis=0, keepdims=True)
    acc[1:2, :] += jnp.sum(a * a, axis=0, keepdims=True)

    @pl.when(i == pl.num_programs(0) - 1)
    def _():
        st1_ref[...] = acc[...]


def _bn_scale(st_ref, g_ref, be_ref, count):
    m = st_ref[0:1, :] / count
    v = st_ref[1:2, :] / count - m * m
    s = g_ref[...] * lax.rsqrt(v + _EPS)
    t = be_ref[...] - m * s
    return s, t


def _folded_layer2(a1_bf, st1_ref, W2_ref, b2_ref, g1_ref, be1_ref, E):
    s1, t1 = _bn_scale(st1_ref, g1_ref, be1_ref, E)
    W2f = (jnp.transpose(s1) * W2_ref[...]).astype(jnp.bfloat16)
    bias = (jnp.dot(t1, W2_ref[...], preferred_element_type=jnp.float32)
            + b2_ref[0:1, :])
    return _leaky(jnp.dot(a1_bf, W2f, preferred_element_type=jnp.float32)
                  + bias)


def _passB_body(E, a1_ref, st1_ref, W2_ref, b2_ref, g1_ref, be1_ref,
                st2_ref, acc):
    i = pl.program_id(0)

    @pl.when(i == 0)
    def _():
        acc[...] = jnp.zeros_like(acc)

    a2 = _folded_layer2(a1_ref[...], st1_ref, W2_ref, b2_ref, g1_ref,
                        be1_ref, E)
    acc[0:1, :] += jnp.sum(a2, axis=0, keepdims=True)
    acc[1:2, :] += jnp.sum(a2 * a2, axis=0, keepdims=True)

    @pl.when(i == pl.num_programs(0) - 1)
    def _():
        st2_ref[...] = acc[...]


def _passC_body(E, a1_ref, xsrc_ref, nd_ref, st1_ref, st2_ref,
                W2_ref, b2_ref, g1_ref, be1_ref,
                W3_ref, b3_ref, g2_ref, be2_ref,
                Wm_ref, bm_ref, out_ref):
    a2 = _folded_layer2(a1_ref[...], st1_ref, W2_ref, b2_ref, g1_ref,
                        be1_ref, E)
    s2, t2 = _bn_scale(st2_ref, g2_ref, be2_ref, E)
    W3f = (jnp.transpose(s2) * W3_ref[...]).astype(jnp.bfloat16)
    bias3 = (jnp.dot(t2, W3_ref[...], preferred_element_type=jnp.float32)
             + b3_ref[0:1, :])
    h3 = jnp.dot(a2.astype(jnp.bfloat16), W3f,
                 preferred_element_type=jnp.float32) + bias3
    G = jnp.dot(xsrc_ref[...].astype(jnp.bfloat16),
                Wm_ref[...].astype(jnp.bfloat16),
                preferred_element_type=jnp.float32) + bm_ref[0:1, :]
    i = pl.program_id(0)
    nb = h3.shape[0]
    dec = _decay_poly(nd_ref[pl.ds(i * nb, nb)])
    out_ref[...] = jnp.reshape(dec, (nb, 1)) * h3 * G


def _row_block(i):
    return (i, 0)


def _pinned(i):
    return (0, 0)


def _edge_passes(xsrc, xdst, es, nd, fi_W1, fi_b1, fi_g1, fi_be1,
                 fi_W2, fi_b2, fi_g2, fi_be2, fi_W3, fi_b3, Wm, bm):
    E, H = xsrc.shape
    nb = E // _BE
    row = pl.BlockSpec((_BE, H), _row_block)
    rowd = pl.BlockSpec((E,), lambda i: (0,))
    full = lambda shape: pl.BlockSpec(shape, _pinned)
    st_shape = jax.ShapeDtypeStruct((8, H), jnp.float32)
    vec = lambda a: a.reshape(1, H)

    a1, st1 = pl.pallas_call(
        _passA_body,
        grid=(nb,),
        in_specs=[row, row, row, full((3 * H, H)), full((1, H))],
        out_specs=[row, full((8, H))],
        out_shape=[jax.ShapeDtypeStruct((E, H), jnp.bfloat16), st_shape],
        scratch_shapes=[pltpu.VMEM((8, H), jnp.float32)],
        interpret=_INTERPRET,
    )(xsrc, es, xdst, fi_W1, vec(fi_b1))

    (st2,) = pl.pallas_call(
        functools.partial(_passB_body, float(E)),
        grid=(nb,),
        in_specs=[row, full((8, H)), full((H, H)), full((1, H)),
                  full((1, H)), full((1, H))],
        out_specs=[full((8, H))],
        out_shape=[st_shape],
        scratch_shapes=[pltpu.VMEM((8, H), jnp.float32)],
        interpret=_INTERPRET,
    )(a1, st1, fi_W2, vec(fi_b2), vec(fi_g1), vec(fi_be1))

    nd3 = nd
    m_st = pl.pallas_call(
        functools.partial(_passC_body, float(E)),
        grid=(nb,),
        in_specs=[row, row, rowd, full((8, H)), full((8, H)),
                  full((H, H)), full((1, H)), full((1, H)), full((1, H)),
                  full((H, H)), full((1, H)), full((1, H)), full((1, H)),
                  full((H, H)), full((1, H))],
        out_specs=row,
        out_shape=jax.ShapeDtypeStruct((E, H), jnp.float32),
        interpret=_INTERPRET,
    )(a1, xsrc, nd3, st1, st2,
      fi_W2, vec(fi_b2), vec(fi_g1), vec(fi_be1),
      fi_W3, vec(fi_b3), vec(fi_g2), vec(fi_be2),
      Wm, vec(bm))
    return m_st


def _node_body(x_ref, inc_ref, Wm_ref, bm_ref, imp_ref,
               W1_ref, b1_ref, g1_ref, be1_ref, W2_ref, b2_ref, out_ref):
    x = x_ref[...]
    n = x.shape[0]
    inc = inc_ref[0] + inc_ref[1]
    mt = imp_ref[...] * (jnp.dot(x, Wm_ref[...], preferred_element_type=jnp.float32)
                         + bm_ref[0:1, :])
    u = mt + inc
    t = _leaky(jnp.dot(u, W1_ref[...], preferred_element_type=jnp.float32)
               + b1_ref[0:1, :])
    m = jnp.sum(t, axis=0, keepdims=True) / n
    v = jnp.sum(t * t, axis=0, keepdims=True) / n - m * m
    t = (t - m) * lax.rsqrt(v + _EPS) * g1_ref[...] + be1_ref[...]
    out_ref[...] = (jnp.dot(t, W2_ref[...], preferred_element_type=jnp.float32)
                    + b2_ref[0:1, :] + x)


def _node_finale(x, parts, Wm, bm, imp_mask, fu_W1, fu_b1, fu_g1, fu_be1,
                 fu_W2, fu_b2):
    N, H = x.shape
    vec = lambda a: a.reshape(1, H)
    return pl.pallas_call(
        _node_body,
        out_shape=jax.ShapeDtypeStruct((N, H), jnp.float32),
        interpret=_INTERPRET,
    )(x, parts, Wm, vec(bm), imp_mask, fu_W1, vec(fu_b1), vec(fu_g1),
      vec(fu_be1), fu_W2, vec(fu_b2))


def kernel(x, edge_index, norm_distance, init_edge_states, Wm, bm, imp_mask,
           fi_W1, fi_b1, fi_g1, fi_be1, fi_W2, fi_b2, fi_g2, fi_be2,
           fi_W3, fi_b3, fu_W1, fu_b1, fu_g1, fu_be1, fu_W2, fu_b2):
    N, H = x.shape
    E = edge_index.shape[1]
    src = edge_index[0]
    dst = edge_index[1]
    xsrc, xdst = _sc_gather(x, src, dst)
    m_st = _edge_passes(xsrc, xdst, init_edge_states, norm_distance,
                        fi_W1, fi_b1, fi_g1, fi_be1,
                        fi_W2, fi_b2, fi_g2, fi_be2,
                        fi_W3, fi_b3, Wm, bm)
    parts = _sc_scatter(m_st, dst, N)
    return _node_finale(x, parts, Wm, bm, imp_mask,
                        fu_W1, fu_b1, fu_g1, fu_be1, fu_W2, fu_b2)


# speedup vs baseline: 3.6957x; 1.0904x over previous
import functools

import jax
import jax.numpy as jnp
from jax import lax
from jax.experimental import pallas as pl
from jax.experimental.pallas import tpu as pltpu
from jax.experimental.pallas import tpu_sc as plsc

_EPS = 1e-5
_INTERPRET = False

_NC = 2
_NS = 16
_CK = 128

_BE = 3200


def _leaky(v):
    return jnp.where(v >= 0, v, 0.01 * v)


def _gather_body(e0, eh, src_hbm, dst_hbm, x_hbm, xsrc_hbm, xdst_hbm,
                 ia0, ia1, ib0, ib1, ra0, ra1, rb0, rb1,
                 sia0, sia1, sib0, sib1, sga0, sga1, sgb0, sgb1):
    nchunks = eh // _CK
    c0 = e0 // _CK
    c = lax.axis_index("c")
    s = lax.axis_index("s")
    wid = s * _NC + c
    nw = _NC * _NS
    iters = pl.cdiv(nchunks, nw)
    IA, IB = (ia0, ia1), (ib0, ib1)
    RA, RB = (ra0, ra1), (rb0, rb1)
    SIA, SIB = (sia0, sia1), (sib0, sib1)
    SGA, SGB = (sga0, sga1), (sgb0, sgb1)

    def chunk_of(k):
        return k * nw + wid

    def issue_idx(k, b):
        gbase = (c0 + chunk_of(k)) * _CK
        pltpu.async_copy(src_hbm.at[pl.ds(gbase, _CK)], IA[b], SIA[b])
        pltpu.async_copy(dst_hbm.at[pl.ds(gbase, _CK)], IB[b], SIB[b])

    def wait_idx(b):
        pltpu.make_async_copy(src_hbm.at[pl.ds(0, _CK)], IA[b], SIA[b]).wait()
        pltpu.make_async_copy(dst_hbm.at[pl.ds(0, _CK)], IB[b], SIB[b]).wait()

    def issue_gather(b):
        pltpu.async_copy(x_hbm.at[IA[b]], RA[b], SGA[b])
        pltpu.async_copy(x_hbm.at[IB[b]], RB[b], SGB[b])

    def wait_gather(b):
        pltpu.make_async_copy(x_hbm.at[pl.ds(0, _CK)], RA[b], SGA[b]).wait()
        pltpu.make_async_copy(x_hbm.at[pl.ds(0, _CK)], RB[b], SGB[b]).wait()

    @pl.when(chunk_of(0) < nchunks)
    def _():
        issue_idx(0, 0)

    @pl.when(chunk_of(1) < nchunks)
    def _():
        issue_idx(1, 1)

    @pl.when(chunk_of(0) < nchunks)
    def _():
        wait_idx(0)
        issue_gather(0)

    def body(q, _):
        for b in (0, 1):
            k = 2 * q + b

            @pl.when(chunk_of(k) < nchunks)
            def _():
                @pl.when(chunk_of(k + 1) < nchunks)
                def _():
                    wait_idx(1 - b)
                    issue_gather(1 - b)

                wait_gather(b)
                base = chunk_of(k) * _CK
                pltpu.sync_copy(RA[b], xsrc_hbm.at[pl.ds(base, _CK)])
                pltpu.sync_copy(RB[b], xdst_hbm.at[pl.ds(base, _CK)])

                @pl.when(chunk_of(k + 2) < nchunks)
                def _():
                    issue_idx(k + 2, b)

        return 0

    lax.fori_loop(0, pl.cdiv(iters, 2), body, 0)


def _sc_gather(x, src, dst, e0, eh):
    H = x.shape[1]
    mesh = plsc.VectorSubcoreMesh(core_axis_name="c", subcore_axis_name="s")
    f = pl.kernel(
        functools.partial(_gather_body, e0, eh),
        out_type=(
            jax.ShapeDtypeStruct((eh, H), jnp.float32),
            jax.ShapeDtypeStruct((eh, H), jnp.float32),
        ),
        mesh=mesh,
        scratch_types=(
            [pltpu.VMEM((_CK,), jnp.int32)] * 4
            + [pltpu.VMEM((_CK, H), jnp.float32)] * 4
            + [pltpu.SemaphoreType.DMA] * 8
        ),
    )
    return f(src, dst, x)


def _scatter_body(e0, dst_hbm, mst_hbm, out_hbm, i0, i1, r0, r1, acc,
                  si0, si1, sr0, sr1):
    eh = mst_hbm.shape[0]
    N = acc.shape[0]
    c = lax.axis_index("c")
    s = lax.axis_index("s")
    e_half = eh // _NC
    nchunks = e_half // _CK
    iters = pl.cdiv(nchunks, _NS)
    I, R = (i0, i1), (r0, r1)
    SI, SR = (si0, si1), (sr0, sr1)

    ntiles_z = 10
    rows_per_tile = N // ntiles_z
    zrows = r0.shape[0]

    def zero_buf(r, _):
        for j in range(8):
            r0[r, pl.ds(j * 16, 16)] = jnp.zeros((16,), jnp.float32)
        return 0

    lax.fori_loop(0, zrows, zero_buf, 0)

    nz = rows_per_tile // zrows
    rem = rows_per_tile - nz * zrows

    @pl.when(s < ntiles_z)
    def _():
        def zero_acc(q, _):
            pltpu.sync_copy(r0,
                            acc.at[pl.ds(s * rows_per_tile + q * zrows, zrows)])
            return 0

        lax.fori_loop(0, nz, zero_acc, 0)
        if rem:
            pltpu.sync_copy(r0.at[pl.ds(0, rem)],
                            acc.at[pl.ds(s * rows_per_tile + nz * zrows, rem)])

    plsc.subcore_barrier()

    def chunk_of(k):
        return k * _NS + s

    def issue_load(k, b):
        base = c * e_half + chunk_of(k) * _CK
        pltpu.async_copy(dst_hbm.at[pl.ds(e0 + base, _CK)], I[b], SI[b])
        pltpu.async_copy(mst_hbm.at[pl.ds(base, _CK)], R[b], SR[b])

    def wait_load(b):
        pltpu.make_async_copy(dst_hbm.at[pl.ds(0, _CK)], I[b], SI[b]).wait()
        pltpu.make_async_copy(mst_hbm.at[pl.ds(0, _CK)], R[b], SR[b]).wait()

    @pl.when(chunk_of(0) < nchunks)
    def _():
        issue_load(0, 0)

    def body(q, _):
        for b in (0, 1):
            k = 2 * q + b

            @pl.when(chunk_of(k) < nchunks)
            def _():
                @pl.when(chunk_of(k + 1) < nchunks)
                def _():
                    issue_load(k + 1, 1 - b)

                wait_load(b)
                pltpu.sync_copy(R[b], acc.at[I[b]], add=True)

        return 0

    lax.fori_loop(0, pl.cdiv(iters, 2), body, 0)

    plsc.subcore_barrier()

    @pl.when(s < ntiles_z)
    def _():
        pltpu.sync_copy(acc.at[pl.ds(s * rows_per_tile, rows_per_tile)],
                        out_hbm.at[c, pl.ds(s * rows_per_tile, rows_per_tile)])


def _sc_scatter(m_st, dst, e0, N):
    H = m_st.shape[1]
    mesh = plsc.VectorSubcoreMesh(core_axis_name="c", subcore_axis_name="s")
    f = pl.kernel(
        functools.partial(_scatter_body, e0),
        out_type=jax.ShapeDtypeStruct((_NC, N, H), jnp.float32),
        mesh=mesh,
        scratch_types=(
            [pltpu.VMEM((_CK,), jnp.int32)] * 2
            + [pltpu.VMEM((_CK, H), jnp.float32)] * 2
            + [pltpu.VMEM_SHARED((N, H), jnp.float32)]
            + [pltpu.SemaphoreType.DMA] * 4
        ),
    )
    return f(dst, m_st)


def _decay_poly(nd):
    t = (jnp.pi / 2.0) * nd
    t2 = t * t
    c = 1.0 / 479001600.0
    for k in (-1.0 / 3628800.0, 1.0 / 40320.0, -1.0 / 720.0, 1.0 / 24.0,
              -0.5, 1.0):
        c = c * t2 + k
    return c


def _passA_body(xsrc_ref, es_ref, xdst_ref, W1_ref, b1_ref,
                a1_ref, st1_ref, acc):
    i = pl.program_id(0)

    @pl.when(i == 0)
    def _():
        acc[...] = jnp.zeros_like(acc)

    h = jnp.concatenate([xsrc_ref[...], es_ref[...], xdst_ref[...]],
                        axis=1).astype(jnp.bfloat16)
    p = jnp.dot(h, W1_ref[...].astype(jnp.bfloat16),
                preferred_element_type=jnp.float32) + b1_ref[0:1, :]
    a = _leaky(p)
    a1_ref[...] = a.astype(jnp.bfloat16)
    acc[0:1, :] += jnp.sum(a, axis=0, keepdims=True)
    acc[1:2, :] += jnp.sum(a * a, axis=0, keepdims=True)

    @pl.when(i == pl.num_programs(0) - 1)
    def _():
        st1_ref[...] = acc[...]


def _bn_scale(st_ref, g_ref, be_ref, count):
    m = st_ref[0:1, :] / count
    v = st_ref[1:2, :] / count - m * m
    s = g_ref[...] * lax.rsqrt(v + _EPS)
    t = be_ref[...] - m * s
    return s, t


def _folded_layer2(a1_bf, st1_ref, W2_ref, b2_ref, g1_ref, be1_ref, E):
    s1, t1 = _bn_scale(st1_ref, g1_ref, be1_ref, E)
    W2f = (jnp.transpose(s1) * W2_ref[...]).astype(jnp.bfloat16)
    bias = (jnp.dot(t1, W2_ref[...], preferred_element_type=jnp.float32)
            + b2_ref[0:1, :])
    return _leaky(jnp.dot(a1_bf, W2f, preferred_element_type=jnp.float32)
                  + bias)


def _passB_body(E, a1_ref, st1_ref, W2_ref, b2_ref, g1_ref, be1_ref,
                st2_ref, acc):
    i = pl.program_id(0)

    @pl.when(i == 0)
    def _():
        acc[...] = jnp.zeros_like(acc)

    a2 = _folded_layer2(a1_ref[...], st1_ref, W2_ref, b2_ref, g1_ref,
                        be1_ref, E)
    acc[0:1, :] += jnp.sum(a2, axis=0, keepdims=True)
    acc[1:2, :] += jnp.sum(a2 * a2, axis=0, keepdims=True)

    @pl.when(i == pl.num_programs(0) - 1)
    def _():
        st2_ref[...] = acc[...]


def _passC_body(E, e0, a1_ref, xsrc_ref, nd_ref, st1_ref, st2_ref,
                W2_ref, b2_ref, g1_ref, be1_ref,
                W3_ref, b3_ref, g2_ref, be2_ref,
                Wm_ref, bm_ref, out_ref):
    a2 = _folded_layer2(a1_ref[...], st1_ref, W2_ref, b2_ref, g1_ref,
                        be1_ref, E)
    s2, t2 = _bn_scale(st2_ref, g2_ref, be2_ref, E)
    W3f = (jnp.transpose(s2) * W3_ref[...]).astype(jnp.bfloat16)
    bias3 = (jnp.dot(t2, W3_ref[...], preferred_element_type=jnp.float32)
             + b3_ref[0:1, :])
    h3 = jnp.dot(a2.astype(jnp.bfloat16), W3f,
                 preferred_element_type=jnp.float32) + bias3
    G = jnp.dot(xsrc_ref[...].astype(jnp.bfloat16),
                Wm_ref[...].astype(jnp.bfloat16),
                preferred_element_type=jnp.float32) + bm_ref[0:1, :]
    i = pl.program_id(0)
    nb = h3.shape[0]
    dec = _decay_poly(nd_ref[pl.ds(e0 + i * nb, nb)])
    out_ref[...] = jnp.reshape(dec, (nb, 1)) * h3 * G


def _row_block(i):
    return (i, 0)


def _pinned(i):
    return (0, 0)


def _pass_a(xsrc, xdst, es, off_b, fi_W1, fi_b1):
    eh, H = xsrc.shape
    nb = eh // _BE
    row = pl.BlockSpec((_BE, H), _row_block)
    esrow = pl.BlockSpec((_BE, H), lambda i: (i + off_b, 0))
    full = lambda shape: pl.BlockSpec(shape, _pinned)
    st_shape = jax.ShapeDtypeStruct((8, H), jnp.float32)
    return pl.pallas_call(
        _passA_body,
        grid=(nb,),
        in_specs=[row, esrow, row, full((3 * H, H)), full((1, H))],
        out_specs=[row, full((8, H))],
        out_shape=[jax.ShapeDtypeStruct((eh, H), jnp.bfloat16), st_shape],
        scratch_shapes=[pltpu.VMEM((8, H), jnp.float32)],
        interpret=_INTERPRET,
    )(xsrc, es, xdst, fi_W1, fi_b1.reshape(1, H))


def _pass_b(a1, st1, E, fi_W2, fi_b2, fi_g1, fi_be1):
    eh, H = a1.shape
    nb = eh // _BE
    row = pl.BlockSpec((_BE, H), _row_block)
    full = lambda shape: pl.BlockSpec(shape, _pinned)
    vec = lambda a: a.reshape(1, H)
    (st2,) = pl.pallas_call(
        functools.partial(_passB_body, float(E)),
        grid=(nb,),
        in_specs=[row, full((8, H)), full((H, H)), full((1, H)),
                  full((1, H)), full((1, H))],
        out_specs=[full((8, H))],
        out_shape=[jax.ShapeDtypeStruct((8, H), jnp.float32)],
        scratch_shapes=[pltpu.VMEM((8, H), jnp.float32)],
        interpret=_INTERPRET,
    )(a1, st1, fi_W2, vec(fi_b2), vec(fi_g1), vec(fi_be1))
    return st2


def _pass_c(a1, xsrc, nd, e0, st1, st2, E,
            fi_W2, fi_b2, fi_g1, fi_be1, fi_W3, fi_b3, fi_g2, fi_be2,
            Wm, bm):
    eh, H = a1.shape
    nb = eh // _BE
    row = pl.BlockSpec((_BE, H), _row_block)
    ndspec = pl.BlockSpec((nd.shape[0],), lambda i: (0,))
    full = lambda shape: pl.BlockSpec(shape, _pinned)
    vec = lambda a: a.reshape(1, H)
    return pl.pallas_call(
        functools.partial(_passC_body, float(E), e0),
        grid=(nb,),
        in_specs=[row, row, ndspec, full((8, H)), full((8, H)),
                  full((H, H)), full((1, H)), full((1, H)), full((1, H)),
                  full((H, H)), full((1, H)), full((1, H)), full((1, H)),
                  full((H, H)), full((1, H))],
        out_specs=row,
        out_shape=jax.ShapeDtypeStruct((eh, H), jnp.float32),
        interpret=_INTERPRET,
    )(a1, xsrc, nd, st1, st2,
      fi_W2, vec(fi_b2), vec(fi_g1), vec(fi_be1),
      fi_W3, vec(fi_b3), vec(fi_g2), vec(fi_be2),
      Wm, vec(bm))


def _node_body(x_ref, p1_ref, p2_ref, Wm_ref, bm_ref, imp_ref,
               W1_ref, b1_ref, g1_ref, be1_ref, W2_ref, b2_ref, out_ref):
    x = x_ref[...]
    n = x.shape[0]
    inc = (p1_ref[0] + p1_ref[1]) + (p2_ref[0] + p2_ref[1])
    mt = imp_ref[...] * (jnp.dot(x, Wm_ref[...], preferred_element_type=jnp.float32)
                         + bm_ref[0:1, :])
    u = mt + inc
    t = _leaky(jnp.dot(u, W1_ref[...], preferred_element_type=jnp.float32)
               + b1_ref[0:1, :])
    m = jnp.sum(t, axis=0, keepdims=True) / n
    v = jnp.sum(t * t, axis=0, keepdims=True) / n - m * m
    t = (t - m) * lax.rsqrt(v + _EPS) * g1_ref[...] + be1_ref[...]
    out_ref[...] = (jnp.dot(t, W2_ref[...], preferred_element_type=jnp.float32)
                    + b2_ref[0:1, :] + x)


def _node_finale(x, parts1, parts2, Wm, bm, imp_mask,
                 fu_W1, fu_b1, fu_g1, fu_be1, fu_W2, fu_b2):
    N, H = x.shape
    vec = lambda a: a.reshape(1, H)
    return pl.pallas_call(
        _node_body,
        out_shape=jax.ShapeDtypeStruct((N, H), jnp.float32),
        interpret=_INTERPRET,
    )(x, parts1, parts2, Wm, vec(bm), imp_mask, fu_W1, vec(fu_b1),
      vec(fu_g1), vec(fu_be1), fu_W2, vec(fu_b2))


def kernel(x, edge_index, norm_distance, init_edge_states, Wm, bm, imp_mask,
           fi_W1, fi_b1, fi_g1, fi_be1, fi_W2, fi_b2, fi_g2, fi_be2,
           fi_W3, fi_b3, fu_W1, fu_b1, fu_g1, fu_be1, fu_W2, fu_b2):
    N, H = x.shape
    E = edge_index.shape[1]
    E2 = E // 2
    src = edge_index[0]
    dst = edge_index[1]

    xsrc1, xdst1 = _sc_gather(x, src, dst, 0, E2)
    xsrc2, xdst2 = _sc_gather(x, src, dst, E2, E2)

    a1_1, p1 = _pass_a(xsrc1, xdst1, init_edge_states, 0, fi_W1, fi_b1)
    a1_2, p2 = _pass_a(xsrc2, xdst2, init_edge_states, E2 // _BE,
                       fi_W1, fi_b1)
    st1 = p1 + p2

    q1 = _pass_b(a1_1, st1, E, fi_W2, fi_b2, fi_g1, fi_be1)
    q2 = _pass_b(a1_2, st1, E, fi_W2, fi_b2, fi_g1, fi_be1)
    st2 = q1 + q2

    m1 = _pass_c(a1_1, xsrc1, norm_distance, 0, st1, st2, E,
                 fi_W2, fi_b2, fi_g1, fi_be1, fi_W3, fi_b3, fi_g2, fi_be2,
                 Wm, bm)
    m2 = _pass_c(a1_2, xsrc2, norm_distance, E2, st1, st2, E,
                 fi_W2, fi_b2, fi_g1, fi_be1, fi_W3, fi_b3, fi_g2, fi_be2,
                 Wm, bm)

    parts1 = _sc_scatter(m1, dst, 0, N)
    parts2 = _sc_scatter(m2, dst, E2, N)

    return _node_finale(x, parts1, parts2, Wm, bm, imp_mask,
                        fu_W1, fu_b1, fu_g1, fu_be1, fu_W2, fu_b2)
